# baseline (speedup 1.0000x reference)
import functools

import numpy as np

import jax
import jax.numpy as jnp
from jax.experimental import pallas as pl
from jax.experimental.pallas import tpu as pltpu

_TN = 2048


def _chamfer_body(x1_ref, x2t_ref, out_ref, d2_scr, s1_scr, *, inv1, inv2):
    i = pl.program_id(1)
    ni = pl.num_programs(1)
    x1 = x1_ref[0]
    x2t = x2t_ref[0]
    sq1 = jnp.sum(x1 * x1, axis=1, keepdims=True)
    sq2 = 0.25 * jnp.sum(x2t * x2t, axis=0, keepdims=True)
    g = jax.lax.dot_general(
        x1, x2t, (((1,), (0,)), ((), ())),
        preferred_element_type=jnp.float32)
    acc = (sq1 + sq2) + g
    p = acc
    while p.shape[1] > 128:
        h = p.shape[1] // 2
        p = jnp.minimum(p[:, :h], p[:, h:])
    d1_tile = jnp.maximum(jnp.min(p.T, axis=0, keepdims=True), 0.0)
    f = d1_tile
    while f.shape[1] > 128:
        h = f.shape[1] // 2
        f = f[:, :h] + f[:, h:]
    part2 = jnp.min(acc, axis=0, keepdims=True)

    @pl.when(i == 0)
    def _init():
        d2_scr[:, :] = part2
        s1_scr[:, :] = f

    @pl.when(i != 0)
    def _accum():
        d2_scr[:, :] = jnp.minimum(d2_scr[:, :], part2)
        s1_scr[:, :] = s1_scr[:, :] + f

    @pl.when(i == ni - 1)
    def _final():
        s1 = jnp.sum(s1_scr[:, :], axis=1, keepdims=True) * inv1
        d2f = jnp.maximum(d2_scr[:, :], 0.0)
        out_ref[0] = s1 + jnp.sum(d2f, axis=1, keepdims=True) * inv2


def _chamfer_local(xyz1, xyz2t, inv1, inv2):
    B, N, D = xyz1.shape
    M = xyz2t.shape[2]
    out = pl.pallas_call(
        functools.partial(_chamfer_body, inv1=inv1, inv2=inv2),
        grid=(B, N // _TN),
        in_specs=[
            pl.BlockSpec((1, _TN, D), lambda b, i: (b, i, 0)),
            pl.BlockSpec((1, D, M), lambda b, i: (b, 0, 0)),
        ],
        out_specs=pl.BlockSpec((1, 1, 1), lambda b, i: (b, 0, 0)),
        out_shape=jax.ShapeDtypeStruct((B, 1, 1), jnp.float32),
        scratch_shapes=[pltpu.VMEM((1, M), jnp.float32),
                        pltpu.VMEM((1, 128), jnp.float32)],
        compiler_params=pltpu.CompilerParams(
            dimension_semantics=("parallel", "arbitrary")),
    )(xyz1, xyz2t)
    return jnp.sum(out)


def kernel(xyz1, xyz2):
    B, N, D = xyz1.shape
    M = xyz2.shape[1]
    xyz2t = -2.0 * jnp.swapaxes(xyz2, 1, 2)
    inv1, inv2 = 1.0 / (B * N), 1.0 / (B * M)
    devs = jax.devices()
    nd = 2 if (len(devs) >= 2 and B % 2 == 0) else 1
    if nd == 1:
        return _chamfer_local(xyz1, xyz2t, inv1, inv2)
    mesh = jax.sharding.Mesh(np.array(devs[:nd]), ("d",))
    P = jax.sharding.PartitionSpec

    def _shard_fn(x1s, x2ts):
        return jax.lax.psum(
            _chamfer_local(x1s, x2ts, inv1, inv2), "d")

    f = jax.shard_map(
        _shard_fn, mesh=mesh,
        in_specs=(P("d"), P("d")), out_specs=P(), check_vma=False)
    return f(xyz1, xyz2t)


# speedup vs baseline: 9.2420x; 9.2420x over previous
import functools

import jax
import jax.numpy as jnp
from jax.experimental import pallas as pl
from jax.experimental.pallas import tpu as pltpu

_TN = 2048


def _chamfer_body(x1_ref, x2t_ref, out_ref, d2_scr, s1_scr, *, inv1, inv2):
    i = pl.program_id(1)
    ni = pl.num_programs(1)
    x1 = x1_ref[0]
    x2t = x2t_ref[0]
    sq1 = jnp.sum(x1 * x1, axis=1, keepdims=True)
    sq2 = 0.25 * jnp.sum(x2t * x2t, axis=0, keepdims=True)
    g = jax.lax.dot_general(
        x1, x2t, (((1,), (0,)), ((), ())),
        preferred_element_type=jnp.float32)
    acc = (sq1 + sq2) + g
    p = acc
    while p.shape[1] > 128:
        h = p.shape[1] // 2
        p = jnp.minimum(p[:, :h], p[:, h:])
    d1_tile = jnp.maximum(jnp.min(p.T, axis=0, keepdims=True), 0.0)
    f = d1_tile
    while f.shape[1] > 128:
        h = f.shape[1] // 2
        f = f[:, :h] + f[:, h:]
    part2 = jnp.min(acc, axis=0, keepdims=True)

    @pl.when(i == 0)
    def _init():
        d2_scr[:, :] = part2
        s1_scr[:, :] = f

    @pl.when(i != 0)
    def _accum():
        d2_scr[:, :] = jnp.minimum(d2_scr[:, :], part2)
        s1_scr[:, :] = s1_scr[:, :] + f

    @pl.when(i == ni - 1)
    def _final():
        s1 = jnp.sum(s1_scr[:, :], axis=1, keepdims=True) * inv1
        d2f = jnp.maximum(d2_scr[:, :], 0.0)
        t = s1 + jnp.sum(d2f, axis=1, keepdims=True) * inv2

        b = pl.program_id(0)

        @pl.when(b == 0)
        def _first():
            out_ref[0] = t

        @pl.when(b != 0)
        def _rest():
            out_ref[0] = out_ref[0] + t


def kernel(xyz1, xyz2):
    B, N, D = xyz1.shape
    M = xyz2.shape[1]
    xyz2t = -2.0 * jnp.swapaxes(xyz2, 1, 2)
    out = pl.pallas_call(
        functools.partial(_chamfer_body,
                          inv1=1.0 / (B * N), inv2=1.0 / (B * M)),
        grid=(B, N // _TN),
        in_specs=[
            pl.BlockSpec((1, _TN, D), lambda b, i: (b, i, 0)),
            pl.BlockSpec((1, D, M), lambda b, i: (b, 0, 0)),
        ],
        out_specs=pl.BlockSpec((1, 1, 1), lambda b, i: (0, 0, 0)),
        out_shape=jax.ShapeDtypeStruct((1, 1, 1), jnp.float32),
        scratch_shapes=[pltpu.VMEM((1, M), jnp.float32),
                        pltpu.VMEM((1, 128), jnp.float32)],
        compiler_params=pltpu.CompilerParams(
            dimension_semantics=("arbitrary", "arbitrary")),
    )(xyz1, xyz2t)
    return out[0, 0, 0]


# speedup vs baseline: 9.4587x; 1.0234x over previous
import functools

import jax
import jax.numpy as jnp
from jax.experimental import pallas as pl
from jax.experimental.pallas import tpu as pltpu

_TN = 2048


def _chamfer_body(x1_ref, x2t_ref, out_ref, *, inv1, inv2):
    b = pl.program_id(0)
    x2t = x2t_ref[0]
    sq2 = 0.25 * jnp.sum(x2t * x2t, axis=0, keepdims=True)
    n = x1_ref.shape[1]

    s1f = None
    d2run = None
    for i in range(n // _TN):
        x1 = x1_ref[0, pl.ds(i * _TN, _TN), :]
        sq1 = jnp.sum(x1 * x1, axis=1, keepdims=True)
        g = jax.lax.dot_general(
            x1, x2t, (((1,), (0,)), ((), ())),
            preferred_element_type=jnp.float32)
        acc = (sq1 + sq2) + g
        p = acc
        while p.shape[1] > 128:
            h = p.shape[1] // 2
            p = jnp.minimum(p[:, :h], p[:, h:])
        d1_tile = jnp.maximum(jnp.min(p.T, axis=0, keepdims=True), 0.0)
        f = d1_tile
        while f.shape[1] > 128:
            h = f.shape[1] // 2
            f = f[:, :h] + f[:, h:]
        part2 = jnp.min(acc, axis=0, keepdims=True)
        s1f = f if s1f is None else s1f + f
        d2run = part2 if d2run is None else jnp.minimum(d2run, part2)

    s1 = jnp.sum(s1f, axis=1, keepdims=True) * inv1
    d2f = jnp.maximum(d2run, 0.0)
    t = s1 + jnp.sum(d2f, axis=1, keepdims=True) * inv2

    @pl.when(b == 0)
    def _first():
        out_ref[0] = t

    @pl.when(b != 0)
    def _rest():
        out_ref[0] = out_ref[0] + t


def kernel(xyz1, xyz2):
    B, N, D = xyz1.shape
    M = xyz2.shape[1]
    xyz2t = -2.0 * jnp.swapaxes(xyz2, 1, 2)
    out = pl.pallas_call(
        functools.partial(_chamfer_body,
                          inv1=1.0 / (B * N), inv2=1.0 / (B * M)),
        grid=(B,),
        in_specs=[
            pl.BlockSpec((1, N, D), lambda b: (b, 0, 0)),
            pl.BlockSpec((1, D, M), lambda b: (b, 0, 0)),
        ],
        out_specs=pl.BlockSpec((1, 1, 1), lambda b: (0, 0, 0)),
        out_shape=jax.ShapeDtypeStruct((1, 1, 1), jnp.float32),
        compiler_params=pltpu.CompilerParams(
            dimension_semantics=("arbitrary",)),
    )(xyz1, xyz2t)
    return out[0, 0, 0]
